# baseline (speedup 1.0000x reference)
import functools

import jax
import jax.numpy as jnp
from jax import lax
from jax.experimental import pallas as pl
from jax.experimental.pallas import tpu as pltpu
from jax.experimental.pallas import tpu_sc as plsc

_NC = 2
_NS = 16
_NW = _NC * _NS
_LANES = 16


def _build_coeff_tables(xs_v, ys_v, sl_v, atab, btab):
    lane = lax.iota(jnp.int32, _LANES)
    r = jnp.minimum(jnp.maximum(lane, 1), 9)
    l = r - 1
    xs_l = plsc.load_gather(xs_v, [l + 1])
    ys_l = plsc.load_gather(ys_v, [l + 1])
    xs_r = plsc.load_gather(xs_v, [r + 1])
    ys_r = plsc.load_gather(ys_v, [r + 1])
    m = (ys_r - ys_l) / (xs_r - xs_l)

    one = jnp.full((_LANES,), 1, jnp.int32)
    s0 = plsc.load_gather(sl_v, [one])
    s1 = plsc.load_gather(sl_v, [one + 1])
    xs0 = plsc.load_gather(xs_v, [one])
    ys0 = plsc.load_gather(ys_v, [one])
    xs9 = plsc.load_gather(xs_v, [one + 9])
    ys9 = plsc.load_gather(ys_v, [one + 9])

    is_left = lane == 0
    is_right = lane >= 10
    bvec = jnp.where(is_left, s0, jnp.where(is_right, s1, m))
    avec = jnp.where(is_left, ys0 - xs0 * s0,
                     jnp.where(is_right, ys9 - xs9 * s1, ys_l - xs_l * m))
    atab[...] = avec
    btab[...] = bvec

    hv = (xs9 - xs0) * (1.0 / 9.0)
    sv = 1.0 / hv
    ov = 1.0 - xs0 * sv
    return sv, ov


@functools.lru_cache(maxsize=None)
def _make_sc_kernel(n, chunk, depth, unroll):
    per_w = n // _NW
    n_chunks = per_w // chunk
    mesh = plsc.VectorSubcoreMesh(core_axis_name="c", subcore_axis_name="s")

    @functools.partial(
        pl.kernel,
        mesh=mesh,
        compiler_params=pltpu.CompilerParams(needs_layout_passes=False),
        out_type=jax.ShapeDtypeStruct((n,), jnp.float32),
        scratch_types=(
            [pltpu.VMEM((_LANES,), jnp.float32)] * 5
            + [pltpu.VMEM((chunk,), jnp.float32)] * (2 * depth)
            + [pltpu.SemaphoreType.DMA] * (2 * depth)
        ),
    )
    def sc_kernel(x_hbm, xs_hbm, ys_hbm, sl_hbm, out_hbm,
                  xs_v, ys_v, sl_v, atab, btab, *bufs):
        inb = bufs[:depth]
        outb = bufs[depth:2 * depth]
        isem = bufs[2 * depth:3 * depth]
        osem = bufs[3 * depth:]
        wid = lax.axis_index("s") * _NC + lax.axis_index("c")
        pltpu.sync_copy(xs_hbm, xs_v)
        pltpu.sync_copy(ys_hbm, ys_v)
        pltpu.sync_copy(sl_hbm, sl_v)
        sv, ov = _build_coeff_tables(xs_v, ys_v, sl_v, atab, btab)

        base = wid * per_w

        def compute(src, dst):
            @plsc.parallel_loop(0, chunk // _LANES, unroll=unroll)
            def _(i):
                xv = src[pl.ds(i * _LANES, _LANES)]
                t = xv * sv + ov
                t = jnp.minimum(jnp.maximum(t, 0.0), 10.0)
                k = t.astype(jnp.int32)
                av = plsc.load_gather(atab, [k])
                bv = plsc.load_gather(btab, [k])
                dst[pl.ds(i * _LANES, _LANES)] = av + bv * xv

        in_h = [None] * depth
        out_h = [None] * depth
        for c in range(min(depth, n_chunks)):
            in_h[c] = pltpu.async_copy(
                x_hbm.at[pl.ds(base + c * chunk, chunk)], inb[c], isem[c])
        for c in range(n_chunks):
            b = c % depth
            in_h[b].wait()
            if c >= depth:
                out_h[b].wait()
            compute(inb[b], outb[b])
            nxt = c + depth
            if nxt < n_chunks:
                in_h[b] = pltpu.async_copy(
                    x_hbm.at[pl.ds(base + nxt * chunk, chunk)], inb[b], isem[b])
            out_h[b] = pltpu.async_copy(
                outb[b], out_hbm.at[pl.ds(base + c * chunk, chunk)], osem[b])
        for c in range(max(0, n_chunks - depth), n_chunks):
            out_h[c % depth].wait()

    return sc_kernel


def kernel(x, xs, slopes, ys):
    shape = x.shape
    xf = x.reshape(-1)
    n = xf.size
    chunk = 8192
    depth = 3
    unroll = 16
    assert n % (_NW * chunk) == 0
    xs16 = jnp.zeros((_LANES,), jnp.float32).at[1 : 1 + xs.size].set(xs)
    ys16 = jnp.zeros((_LANES,), jnp.float32).at[1 : 1 + ys.size].set(ys)
    sl16 = jnp.zeros((_LANES,), jnp.float32).at[1 : 1 + slopes.size].set(slopes)
    out = _make_sc_kernel(n, chunk, depth, unroll)(xf, xs16, ys16, sl16)
    return out.reshape(shape)
